# baseline (speedup 1.0000x reference)
import functools

import jax
import jax.numpy as jnp
from jax import lax
from jax.experimental import pallas as pl
from jax.experimental.pallas import tpu as pltpu
from jax.experimental.pallas import tpu_sc as plsc


def _make_sc_kernel(B, F, b_per_w, NC):
    mesh = plsc.VectorSubcoreMesh(core_axis_name="c", subcore_axis_name="s")

    @functools.partial(
        pl.kernel,
        mesh=mesh,
        out_type=jax.ShapeDtypeStruct((B,), jnp.float32),
        scratch_types=[
            pltpu.VMEM((F, b_per_w), jnp.int32),
            pltpu.VMEM((F, b_per_w), jnp.float32),
            pltpu.VMEM((b_per_w,), jnp.float32),
            pltpu.VMEM((16,), jnp.float32),
            pltpu.SemaphoreType.DMA,
            pltpu.SemaphoreType.DMA,
            pltpu.SemaphoreType.DMA,
            pltpu.SemaphoreType.DMA,
            pltpu.SemaphoreType.DMA,
        ],
    )
    def sc_k(xT_hbm, table_hbm, bias_hbm, out_hbm, idx_v, rows_v, out_v,
             bias_v, sem, bsem, csem, dsem, esem):
        wid = lax.axis_index("s") * NC + lax.axis_index("c")
        base = wid * b_per_w
        half = F // 2
        tbl = table_hbm.at[0]
        pltpu.sync_copy(xT_hbm.at[:, pl.ds(base, b_per_w)], idx_v)

        def fire(f, grp_sem):
            pltpu.async_copy(tbl.at[idx_v.at[f]], rows_v.at[f], grp_sem)

        groups = [(0, 8, sem), (8, 16, bsem), (16, 24, dsem), (24, F, esem)]
        for lo, hi, gs in groups:
            lax.fori_loop(lo, hi, lambda f, _, gs=gs: (fire(f, gs), 0)[1], 0)
        bias_cp = pltpu.async_copy(bias_hbm, bias_v.at[pl.ds(0, 1)], csem)

        n_ch = b_per_w // 16

        def drain(grp_sem, n):
            def body(_, carry):
                pltpu.make_async_copy(
                    tbl.at[idx_v.at[0]], rows_v.at[0], grp_sem
                ).wait()
                return carry

            lax.fori_loop(0, n, body, 0)

        def accumulate(lo, hi, accs):
            def body(f, accs):
                return tuple(
                    accs[c] + rows_v[f, pl.ds(c * 16, 16)]
                    for c in range(n_ch)
                )

            return lax.fori_loop(lo, hi, body, accs)

        zeros = jnp.zeros((16,), jnp.float32)
        accs = (zeros,) * n_ch
        for lo, hi, gs in groups:
            drain(gs, hi - lo)
            accs = accumulate(lo, hi, accs)
        bias_cp.wait()
        b = bias_v[...][0]
        for c in range(n_ch):
            out_v[pl.ds(c * 16, 16)] = accs[c] + b
        pltpu.sync_copy(out_v, out_hbm.at[pl.ds(base, b_per_w)])

    return sc_k


def kernel(x, fc_weight, bias):
    B, F = x.shape
    info = plsc.get_sparse_core_info()
    NC, NS = info.num_cores, info.num_subcores
    NW = NC * NS
    b_per_w = B // NW

    xT = x.astype(jnp.int32).T
    table = fc_weight.reshape(1, -1)

    sc_k = _make_sc_kernel(B, F, b_per_w, NC)
    out = sc_k(xT, table, bias)
    return out.reshape(B, 1)


# speedup vs baseline: 1.0250x; 1.0250x over previous
import functools

import jax
import jax.numpy as jnp
from jax import lax
from jax.experimental import pallas as pl
from jax.experimental.pallas import tpu as pltpu
from jax.experimental.pallas import tpu_sc as plsc


def _make_sc_kernel(B, F, b_per_w, NC):
    mesh = plsc.VectorSubcoreMesh(core_axis_name="c", subcore_axis_name="s")

    @functools.partial(
        pl.kernel,
        mesh=mesh,
        out_type=jax.ShapeDtypeStruct((B,), jnp.float32),
        scratch_types=[
            pltpu.VMEM((F, b_per_w), jnp.int32),
            pltpu.VMEM((F, b_per_w), jnp.float32),
            pltpu.VMEM((b_per_w,), jnp.float32),
            pltpu.VMEM((16,), jnp.float32),
            pltpu.SemaphoreType.DMA,
            pltpu.SemaphoreType.DMA,
            pltpu.SemaphoreType.DMA,
        ],
    )
    def sc_k(xT_hbm, table_hbm, bias_hbm, out_hbm, idx_v, rows_v, out_v,
             bias_v, sem, bsem, csem):
        wid = lax.axis_index("s") * NC + lax.axis_index("c")
        base = wid * b_per_w
        half = F // 2
        tbl = table_hbm.at[0]
        pltpu.sync_copy(xT_hbm.at[:, pl.ds(base, b_per_w)], idx_v)

        def fire(f, grp_sem):
            pltpu.async_copy(tbl.at[idx_v.at[f]], rows_v.at[f], grp_sem)

        groups = [(0, half, sem), (half, F, bsem)]
        for lo, hi, gs in groups:
            lax.fori_loop(lo, hi, lambda f, _, gs=gs: (fire(f, gs), 0)[1], 0)
        bias_cp = pltpu.async_copy(bias_hbm, bias_v.at[pl.ds(0, 1)], csem)

        n_ch = b_per_w // 16

        def drain(grp_sem, n):
            def body(_, carry):
                pltpu.make_async_copy(
                    tbl.at[idx_v.at[0]], rows_v.at[0], grp_sem
                ).wait()
                return carry

            lax.fori_loop(0, n, body, 0)

        def accumulate(lo, hi, accs):
            def body(f, accs):
                return tuple(
                    accs[c] + rows_v[f, pl.ds(c * 16, 16)]
                    for c in range(n_ch)
                )

            return lax.fori_loop(lo, hi, body, accs)

        zeros = jnp.zeros((16,), jnp.float32)
        accs = (zeros,) * n_ch
        for lo, hi, gs in groups:
            drain(gs, hi - lo)
            accs = accumulate(lo, hi, accs)
        bias_cp.wait()
        b = bias_v[...][0]
        for c in range(n_ch):
            out_v[pl.ds(c * 16, 16)] = accs[c] + b
        pltpu.sync_copy(out_v, out_hbm.at[pl.ds(base, b_per_w)])

    return sc_k


def kernel(x, fc_weight, bias):
    B, F = x.shape
    info = plsc.get_sparse_core_info()
    NC, NS = info.num_cores, info.num_subcores
    NW = NC * NS
    b_per_w = B // NW

    xT = x.astype(jnp.int32).T
    table = fc_weight.reshape(1, -1)

    sc_k = _make_sc_kernel(B, F, b_per_w, NC)
    out = sc_k(xT, table, bias)
    return out.reshape(B, 1)


# speedup vs baseline: 1.0359x; 1.0106x over previous
import functools

import jax
import jax.numpy as jnp
from jax import lax
from jax.experimental import pallas as pl
from jax.experimental.pallas import tpu as pltpu
from jax.experimental.pallas import tpu_sc as plsc


def _make_sc_kernel(B, F, b_per_w, NC):
    mesh = plsc.VectorSubcoreMesh(core_axis_name="c", subcore_axis_name="s")

    @functools.partial(
        pl.kernel,
        mesh=mesh,
        out_type=jax.ShapeDtypeStruct((B,), jnp.float32),
        scratch_types=[
            pltpu.VMEM((F, b_per_w), jnp.int32),
            pltpu.VMEM((F, b_per_w), jnp.float32),
            pltpu.VMEM((b_per_w,), jnp.float32),
            pltpu.VMEM((16,), jnp.float32),
            pltpu.SemaphoreType.DMA((F,)),
            pltpu.SemaphoreType.DMA,
        ],
    )
    def sc_k(xT_hbm, table_hbm, bias_hbm, out_hbm, idx_v, rows_v, out_v,
             bias_v, sems, csem):
        wid = lax.axis_index("s") * NC + lax.axis_index("c")
        base = wid * b_per_w
        half = F // 2
        tbl = table_hbm.at[0]
        pltpu.sync_copy(xT_hbm.at[:, pl.ds(base, b_per_w)], idx_v)

        def fire(f, _):
            pltpu.async_copy(tbl.at[idx_v.at[f]], rows_v.at[f], sems.at[f])
            return 0

        lax.fori_loop(0, F, fire, 0)
        bias_cp = pltpu.async_copy(bias_hbm, bias_v.at[pl.ds(0, 1)], csem)

        n_ch = b_per_w // 16

        def body(f, accs):
            pltpu.make_async_copy(
                tbl.at[idx_v.at[0]], rows_v.at[f], sems.at[f]
            ).wait()
            return tuple(
                accs[c] + rows_v[f, pl.ds(c * 16, 16)] for c in range(n_ch)
            )

        zeros = jnp.zeros((16,), jnp.float32)
        accs = lax.fori_loop(0, F, body, (zeros,) * n_ch)
        bias_cp.wait()
        b = bias_v[...][0]
        for c in range(n_ch):
            out_v[pl.ds(c * 16, 16)] = accs[c] + b
        pltpu.sync_copy(out_v, out_hbm.at[pl.ds(base, b_per_w)])

    return sc_k


def kernel(x, fc_weight, bias):
    B, F = x.shape
    info = plsc.get_sparse_core_info()
    NC, NS = info.num_cores, info.num_subcores
    NW = NC * NS
    b_per_w = B // NW

    xT = x.astype(jnp.int32).T
    table = fc_weight.reshape(1, -1)

    sc_k = _make_sc_kernel(B, F, b_per_w, NC)
    out = sc_k(xT, table, bias)
    return out.reshape(B, 1)
